# Initial kernel scaffold; baseline (speedup 1.0000x reference)
import jax
import jax.numpy as jnp
from jax import lax
from jax.experimental import pallas as pl
from jax.experimental.pallas import tpu as pltpu
from jax.experimental.pallas import tpu_sc as plsc

_N = 10000
_E = 320000
_H = 128
_NC = 2
_NS = 16
_NW = _NC * _NS
_CH = 128
_NCHUNK = _E // _CH
_CPW = _NCHUNK // _NW
_CREM = _NCHUNK % _NW
_RPT = 625
_NB = 2000
_EB = 2000

_f32 = jnp.float32


def _full(shape):
    return pl.BlockSpec(shape, lambda i: (0,) * len(shape))


def _rows(shape):
    return pl.BlockSpec(shape, lambda i: (i,) + (0,) * (len(shape) - 1))


def _relu(v):
    return jnp.maximum(v, 0.0)


def _enc_node_body(x_ref, a1, b1, a2, b2, w1a, w1b, bm1, h_out, p_out, q_out):
    t = _relu(x_ref[...] @ a1[...] + b1[...])
    h = t @ a2[...] + b2[...]
    h_out[...] = h
    p_out[...] = h @ w1a[...] + bm1[...]
    q_out[...] = h @ w1b[...]


def _enc_edge_body(ea_ref, a1, b1, a2, b2, e_out):
    t = _relu(ea_ref[...] @ a1[...] + b1[...])
    e_out[...] = t @ a2[...] + b2[...]


def _edge_body(s1_ref, s2_ref, e_ref, w1c, w2, b2, m_out):
    pre = s1_ref[...] + s2_ref[...] + e_ref[...] @ w1c[...]
    m_out[...] = _relu(pre) @ w2[...] + b2[...]


def _node_body(h_ref, a0_ref, a1_ref, u1a, u1b, c1, u2, c2, w1a, w1b, bm1,
               h_out, p_out, q_out):
    h = h_ref[...]
    agg = a0_ref[...] + a1_ref[...]
    z = _relu(h @ u1a[...] + agg @ u1b[...] + c1[...])
    hn = h + z @ u2[...] + c2[...]
    h_out[...] = hn
    p_out[...] = hn @ w1a[...] + bm1[...]
    q_out[...] = hn @ w1b[...]


def _node_final_body(h_ref, a0_ref, a1_ref, u1a, u1b, c1, u2, c2,
                     d1, g1, d2, g2, d3, g3, msk_ref, out_ref):
    h = h_ref[...]
    agg = a0_ref[...] + a1_ref[...]
    z = _relu(h @ u1a[...] + agg @ u1b[...] + c1[...])
    hn = h + z @ u2[...] + c2[...]
    z1 = _relu(hn @ d1[...] + g1[...])
    z2 = _relu(z1 @ d2[...] + g2[...])
    out_ref[...] = (z2 @ d3[...] + g3[...]) * msk_ref[...]


def _nmat(n, d):
    return jax.ShapeDtypeStruct((n, d), _f32)


def _encode_nodes(x2, a1, b1, a2, b2, w1a, w1b, bm1):
    return pl.pallas_call(
        _enc_node_body,
        grid=(_N // _NB,),
        in_specs=[_rows((_NB, x2.shape[1])), _full(a1.shape), _full(b1.shape),
                  _full(a2.shape), _full(b2.shape), _full(w1a.shape),
                  _full(w1b.shape), _full(bm1.shape)],
        out_specs=[_rows((_NB, _H))] * 3,
        out_shape=[_nmat(_N, _H)] * 3,
    )(x2, a1, b1, a2, b2, w1a, w1b, bm1)


def _encode_edges(ea, a1, b1, a2, b2):
    return pl.pallas_call(
        _enc_edge_body,
        grid=(_E // _EB,),
        in_specs=[_rows((_EB, ea.shape[1])), _full(a1.shape), _full(b1.shape),
                  _full(a2.shape), _full(b2.shape)],
        out_specs=_rows((_EB, _H)),
        out_shape=_nmat(_E, _H),
    )(ea, a1, b1, a2, b2)


def _edge_mlp(s1, s2, e, w1c, w2, b2):
    return pl.pallas_call(
        _edge_body,
        grid=(_E // _EB,),
        in_specs=[_rows((_EB, _H))] * 3 + [_full(w1c.shape), _full(w2.shape),
                                           _full(b2.shape)],
        out_specs=_rows((_EB, _H)),
        out_shape=_nmat(_E, _H),
    )(s1, s2, e, w1c, w2, b2)


def _node_update(h, a0, a1m, u1a, u1b, c1, u2, c2, w1a, w1b, bm1):
    return pl.pallas_call(
        _node_body,
        grid=(_N // _NB,),
        in_specs=[_rows((_NB, _H))] * 3 +
                 [_full(w.shape) for w in (u1a, u1b, c1, u2, c2, w1a, w1b, bm1)],
        out_specs=[_rows((_NB, _H))] * 3,
        out_shape=[_nmat(_N, _H)] * 3,
    )(h, a0, a1m, u1a, u1b, c1, u2, c2, w1a, w1b, bm1)


def _node_final(h, a0, a1m, u1a, u1b, c1, u2, c2, d1, g1, d2, g2, d3, g3, msk):
    return pl.pallas_call(
        _node_final_body,
        grid=(_N // _NB,),
        in_specs=[_rows((_NB, _H))] * 3 +
                 [_full(w.shape) for w in (u1a, u1b, c1, u2, c2,
                                           d1, g1, d2, g2, d3, g3)] +
                 [_rows((_NB, 3))],
        out_specs=_rows((_NB, 3)),
        out_shape=_nmat(_N, 3),
    )(h, a0, a1m, u1a, u1b, c1, u2, c2, d1, g1, d2, g2, d3, g3, msk)


def _sc_mesh():
    return plsc.VectorSubcoreMesh(core_axis_name="c", subcore_axis_name="s",
                                  num_cores=_NC, num_subcores=_NS)


def _worker_id():
    return lax.axis_index("s") * _NC + lax.axis_index("c")


def _nchunks(w):
    return jnp.where(w < _CREM, _CPW + 1, _CPW)


def _sc_gather_body(p_hbm, q_hbm, src_hbm, dst_hbm, s1_hbm, s2_hbm,
                    idx_v, buf_v, sem):
    w = _worker_id()

    def step(j, carry):
        base = (w + j * _NW) * _CH
        pltpu.sync_copy(src_hbm.at[pl.ds(base, _CH)], idx_v)
        pltpu.async_copy(p_hbm.at[idx_v], buf_v, sem).wait()
        pltpu.sync_copy(buf_v, s1_hbm.at[pl.ds(base, _CH)])
        pltpu.sync_copy(dst_hbm.at[pl.ds(base, _CH)], idx_v)
        pltpu.async_copy(q_hbm.at[idx_v], buf_v, sem).wait()
        pltpu.sync_copy(buf_v, s2_hbm.at[pl.ds(base, _CH)])
        return carry

    lax.fori_loop(0, _nchunks(w), step, 0)


def _sc_gather(p, q, src, dst):
    fn = pl.kernel(
        _sc_gather_body,
        out_type=[_nmat(_E, _H), _nmat(_E, _H)],
        mesh=_sc_mesh(),
        scratch_types=[pltpu.VMEM((_CH,), jnp.int32),
                       pltpu.VMEM((_CH, _H), _f32),
                       pltpu.SemaphoreType.DMA],
    )
    return fn(p, q, src, dst)


def _sc_scatter_body(m_hbm, dst_hbm, z_hbm, out_hbm, idx_v, buf_v, agg_sh, sem):
    c = lax.axis_index("c")
    s = lax.axis_index("s")
    w = s * _NC + c
    r0 = s * _RPT
    pltpu.sync_copy(z_hbm.at[pl.ds(r0, _RPT)], agg_sh.at[pl.ds(r0, _RPT)])
    plsc.subcore_barrier()

    def step(j, carry):
        base = (w + j * _NW) * _CH
        pltpu.sync_copy(dst_hbm.at[pl.ds(base, _CH)], idx_v)
        pltpu.sync_copy(m_hbm.at[pl.ds(base, _CH)], buf_v)
        pltpu.sync_copy(buf_v, agg_sh.at[idx_v], add=True)
        return carry

    lax.fori_loop(0, _nchunks(w), step, 0)
    plsc.subcore_barrier()
    pltpu.sync_copy(agg_sh.at[pl.ds(r0, _RPT)], out_hbm.at[c, pl.ds(r0, _RPT)])


def _sc_scatter(m, dst, zeros_n):
    fn = pl.kernel(
        _sc_scatter_body,
        out_type=jax.ShapeDtypeStruct((_NC, _N, _H), _f32),
        mesh=_sc_mesh(),
        scratch_types=[pltpu.VMEM((_CH,), jnp.int32),
                       pltpu.VMEM((_CH, _H), _f32),
                       pltpu.VMEM_SHARED((_N, _H), _f32),
                       pltpu.SemaphoreType.DMA],
    )
    return fn(m, dst, zeros_n)


def _r1(b):
    return b.reshape(1, -1)


def kernel(x, coords, edge_attr, bc_disp, bc_rot, edge_index,
           enc_node, enc_edge, mp_params, dec):
    x2 = jnp.concatenate([coords, x[:, 3:]], axis=1)
    src = edge_index[0]
    dst = edge_index[1]
    mask3 = jnp.concatenate([1.0 - bc_disp, 1.0 - bc_disp, 1.0 - bc_rot],
                            axis=1)
    zeros_n = jnp.zeros((_N, _H), _f32)

    (ne1, nb1), (ne2, nb2) = enc_node
    (ee1, eb1), (ee2, eb2) = enc_edge

    def _w1_split(l):
        w1, b1 = mp_params[l][0][0]
        return w1[:_H], w1[_H:2 * _H], w1[2 * _H:], b1

    e = _encode_edges(edge_attr, ee1, _r1(eb1), ee2, _r1(eb2))

    w1a, w1b, _, b1 = _w1_split(0)
    h, p, q = _encode_nodes(x2, ne1, _r1(nb1), ne2, _r1(nb2),
                            w1a, w1b, _r1(b1))

    pred = None
    for l in range(len(mp_params)):
        edge_mlp, node_mlp = mp_params[l]
        _, (w2, b2) = edge_mlp
        (u1, c1), (u2, c2) = node_mlp
        _, _, w1c, _ = _w1_split(l)

        s1, s2 = _sc_gather(p, q, src, dst)
        m = _edge_mlp(s1, s2, e, w1c, w2, _r1(b2))
        aggs = _sc_scatter(m, dst, zeros_n)

        u1a, u1b = u1[:_H], u1[_H:]
        if l + 1 < len(mp_params):
            w1a_n, w1b_n, _, b1_n = _w1_split(l + 1)
            h, p, q = _node_update(h, aggs[0], aggs[1], u1a, u1b, _r1(c1),
                                   u2, _r1(c2), w1a_n, w1b_n, _r1(b1_n))
        else:
            (d1, g1), (d2, g2), (d3, g3) = dec
            pred = _node_final(h, aggs[0], aggs[1], u1a, u1b, _r1(c1),
                               u2, _r1(c2), d1, _r1(g1), d2, _r1(g2),
                               d3, _r1(g3), mask3)
    return pred


# speedup vs baseline: 2.4628x; 2.4628x over previous
import jax
import jax.numpy as jnp
from jax import lax
from jax.experimental import pallas as pl
from jax.experimental.pallas import tpu as pltpu
from jax.experimental.pallas import tpu_sc as plsc

_N = 10000
_E = 320000
_H = 128
_NC = 2
_NS = 16
_NW = _NC * _NS
_CH = 128
_NCHUNK = _E // _CH
_CPW = _NCHUNK // _NW
_CREM = _NCHUNK % _NW
_RPT = 624
_RLAST0 = _RPT * 15
_RLAST = _N - _RLAST0
_NB = 2000
_EB = 2000

_f32 = jnp.float32


def _full(shape):
    return pl.BlockSpec(shape, lambda i: (0,) * len(shape))


def _rows(shape):
    return pl.BlockSpec(shape, lambda i: (i,) + (0,) * (len(shape) - 1))


def _relu(v):
    return jnp.maximum(v, 0.0)


def _enc_node_body(x_ref, a1, b1, a2, b2, w1a, w1b, bm1, h_out, p_out, q_out):
    t = _relu(x_ref[...] @ a1[...] + b1[...])
    h = t @ a2[...] + b2[...]
    h_out[...] = h
    p_out[...] = h @ w1a[...] + bm1[...]
    q_out[...] = h @ w1b[...]


def _enc_edge_body(ea_ref, a1, b1, a2, b2, e_out):
    t = _relu(ea_ref[...] @ a1[...] + b1[...])
    e_out[...] = t @ a2[...] + b2[...]


def _edge_body(s1_ref, s2_ref, e_ref, w1c, w2, b2, m_out):
    pre = s1_ref[...] + s2_ref[...] + e_ref[...] @ w1c[...]
    m_out[...] = _relu(pre) @ w2[...] + b2[...]


def _node_body(h_ref, a0_ref, a1_ref, u1a, u1b, c1, u2, c2, w1a, w1b, bm1,
               h_out, p_out, q_out):
    h = h_ref[...]
    agg = a0_ref[...] + a1_ref[...]
    z = _relu(h @ u1a[...] + agg @ u1b[...] + c1[...])
    hn = h + z @ u2[...] + c2[...]
    h_out[...] = hn
    p_out[...] = hn @ w1a[...] + bm1[...]
    q_out[...] = hn @ w1b[...]


def _node_final_body(h_ref, a0_ref, a1_ref, u1a, u1b, c1, u2, c2,
                     d1, g1, d2, g2, d3, g3, msk_ref, out_ref):
    h = h_ref[...]
    agg = a0_ref[...] + a1_ref[...]
    z = _relu(h @ u1a[...] + agg @ u1b[...] + c1[...])
    hn = h + z @ u2[...] + c2[...]
    z1 = _relu(hn @ d1[...] + g1[...])
    z2 = _relu(z1 @ d2[...] + g2[...])
    out_ref[...] = (z2 @ d3[...] + g3[...]) * msk_ref[...]


def _nmat(n, d):
    return jax.ShapeDtypeStruct((n, d), _f32)


def _encode_nodes(x2, a1, b1, a2, b2, w1a, w1b, bm1):
    return pl.pallas_call(
        _enc_node_body,
        grid=(_N // _NB,),
        in_specs=[_rows((_NB, x2.shape[1])), _full(a1.shape), _full(b1.shape),
                  _full(a2.shape), _full(b2.shape), _full(w1a.shape),
                  _full(w1b.shape), _full(bm1.shape)],
        out_specs=[_rows((_NB, _H))] * 3,
        out_shape=[_nmat(_N, _H)] * 3,
    )(x2, a1, b1, a2, b2, w1a, w1b, bm1)


def _encode_edges(ea, a1, b1, a2, b2):
    return pl.pallas_call(
        _enc_edge_body,
        grid=(_E // _EB,),
        in_specs=[_rows((_EB, ea.shape[1])), _full(a1.shape), _full(b1.shape),
                  _full(a2.shape), _full(b2.shape)],
        out_specs=_rows((_EB, _H)),
        out_shape=_nmat(_E, _H),
    )(ea, a1, b1, a2, b2)


def _edge_mlp(s1, s2, e, w1c, w2, b2):
    return pl.pallas_call(
        _edge_body,
        grid=(_E // _EB,),
        in_specs=[_rows((_EB, _H))] * 3 + [_full(w1c.shape), _full(w2.shape),
                                           _full(b2.shape)],
        out_specs=_rows((_EB, _H)),
        out_shape=_nmat(_E, _H),
    )(s1, s2, e, w1c, w2, b2)


def _node_update(h, a0, a1m, u1a, u1b, c1, u2, c2, w1a, w1b, bm1):
    return pl.pallas_call(
        _node_body,
        grid=(_N // _NB,),
        in_specs=[_rows((_NB, _H))] * 3 +
                 [_full(w.shape) for w in (u1a, u1b, c1, u2, c2, w1a, w1b, bm1)],
        out_specs=[_rows((_NB, _H))] * 3,
        out_shape=[_nmat(_N, _H)] * 3,
    )(h, a0, a1m, u1a, u1b, c1, u2, c2, w1a, w1b, bm1)


def _node_final(h, a0, a1m, u1a, u1b, c1, u2, c2, d1, g1, d2, g2, d3, g3, msk):
    return pl.pallas_call(
        _node_final_body,
        grid=(_N // _NB,),
        in_specs=[_rows((_NB, _H))] * 3 +
                 [_full(w.shape) for w in (u1a, u1b, c1, u2, c2,
                                           d1, g1, d2, g2, d3, g3)] +
                 [_rows((_NB, 3))],
        out_specs=_rows((_NB, 3)),
        out_shape=_nmat(_N, 3),
    )(h, a0, a1m, u1a, u1b, c1, u2, c2, d1, g1, d2, g2, d3, g3, msk)


def _sc_mesh():
    return plsc.VectorSubcoreMesh(core_axis_name="c", subcore_axis_name="s",
                                  num_cores=_NC, num_subcores=_NS)


def _worker_id():
    return lax.axis_index("s") * _NC + lax.axis_index("c")


def _nchunks(w):
    return jnp.where(w < _CREM, _CPW + 1, _CPW)


def _sc_gather_body(p_hbm, q_hbm, src_hbm, dst_hbm, s1_hbm, s2_hbm,
                    idx_v, buf_v, sem):
    w = _worker_id()

    def step(j, carry):
        base = (w + j * _NW) * _CH
        pltpu.sync_copy(src_hbm.at[pl.ds(base, _CH)], idx_v)
        pltpu.async_copy(p_hbm.at[idx_v], buf_v, sem).wait()
        pltpu.sync_copy(buf_v, s1_hbm.at[pl.ds(base, _CH)])
        pltpu.sync_copy(dst_hbm.at[pl.ds(base, _CH)], idx_v)
        pltpu.async_copy(q_hbm.at[idx_v], buf_v, sem).wait()
        pltpu.sync_copy(buf_v, s2_hbm.at[pl.ds(base, _CH)])
        return carry

    lax.fori_loop(0, _nchunks(w), step, 0)


def _sc_gather(p, q, src, dst):
    fn = pl.kernel(
        _sc_gather_body,
        out_type=[_nmat(_E, _H), _nmat(_E, _H)],
        mesh=_sc_mesh(),
        scratch_types=[pltpu.VMEM((_CH,), jnp.int32),
                       pltpu.VMEM((_CH, _H), _f32),
                       pltpu.SemaphoreType.DMA],
    )
    return fn(p, q, src, dst)


def _sc_scatter_body(m_hbm, dst_hbm, z_hbm, out_hbm, idx_v, buf_v, agg_sh, sem):
    c = lax.axis_index("c")
    s = lax.axis_index("s")
    w = s * _NC + c
    r0 = s * _RPT
    @pl.when(s < _NS - 1)
    def _():
        pltpu.sync_copy(z_hbm.at[pl.ds(r0, _RPT)], agg_sh.at[pl.ds(r0, _RPT)])

    @pl.when(s == _NS - 1)
    def _():
        pltpu.sync_copy(z_hbm.at[pl.ds(_RLAST0, _RLAST)],
                        agg_sh.at[pl.ds(_RLAST0, _RLAST)])

    plsc.subcore_barrier()

    def step(j, carry):
        base = (w + j * _NW) * _CH
        pltpu.sync_copy(dst_hbm.at[pl.ds(base, _CH)], idx_v)
        pltpu.sync_copy(m_hbm.at[pl.ds(base, _CH)], buf_v)
        pltpu.sync_copy(buf_v, agg_sh.at[idx_v], add=True)
        return carry

    lax.fori_loop(0, _nchunks(w), step, 0)
    plsc.subcore_barrier()

    @pl.when(s < _NS - 1)
    def _():
        pltpu.sync_copy(agg_sh.at[pl.ds(r0, _RPT)],
                        out_hbm.at[c, pl.ds(r0, _RPT)])

    @pl.when(s == _NS - 1)
    def _():
        pltpu.sync_copy(agg_sh.at[pl.ds(_RLAST0, _RLAST)],
                        out_hbm.at[c, pl.ds(_RLAST0, _RLAST)])


def _sc_scatter(m, dst, zeros_n):
    fn = pl.kernel(
        _sc_scatter_body,
        out_type=jax.ShapeDtypeStruct((_NC, _N, _H), _f32),
        mesh=_sc_mesh(),
        scratch_types=[pltpu.VMEM((_CH,), jnp.int32),
                       pltpu.VMEM((_CH, _H), _f32),
                       pltpu.VMEM_SHARED((_N, _H), _f32),
                       pltpu.SemaphoreType.DMA],
    )
    return fn(m, dst, zeros_n)


def _r1(b):
    return b.reshape(1, -1)


def kernel(x, coords, edge_attr, bc_disp, bc_rot, edge_index,
           enc_node, enc_edge, mp_params, dec):
    x2 = jnp.concatenate([coords, x[:, 3:]], axis=1)
    src = edge_index[0]
    dst = edge_index[1]
    mask3 = jnp.concatenate([1.0 - bc_disp, 1.0 - bc_disp, 1.0 - bc_rot],
                            axis=1)
    zeros_n = jnp.zeros((_N, _H), _f32)

    (ne1, nb1), (ne2, nb2) = enc_node
    (ee1, eb1), (ee2, eb2) = enc_edge

    def _w1_split(l):
        w1, b1 = mp_params[l][0][0]
        return w1[:_H], w1[_H:2 * _H], w1[2 * _H:], b1

    e = _encode_edges(edge_attr, ee1, _r1(eb1), ee2, _r1(eb2))

    w1a, w1b, _, b1 = _w1_split(0)
    h, p, q = _encode_nodes(x2, ne1, _r1(nb1), ne2, _r1(nb2),
                            w1a, w1b, _r1(b1))

    pred = None
    for l in range(len(mp_params)):
        edge_mlp, node_mlp = mp_params[l]
        _, (w2, b2) = edge_mlp
        (u1, c1), (u2, c2) = node_mlp
        _, _, w1c, _ = _w1_split(l)

        s1, s2 = _sc_gather(p, q, src, dst)
        m = _edge_mlp(s1, s2, e, w1c, w2, _r1(b2))
        aggs = _sc_scatter(m, dst, zeros_n)

        u1a, u1b = u1[:_H], u1[_H:]
        if l + 1 < len(mp_params):
            w1a_n, w1b_n, _, b1_n = _w1_split(l + 1)
            h, p, q = _node_update(h, aggs[0], aggs[1], u1a, u1b, _r1(c1),
                                   u2, _r1(c2), w1a_n, w1b_n, _r1(b1_n))
        else:
            (d1, g1), (d2, g2), (d3, g3) = dec
            pred = _node_final(h, aggs[0], aggs[1], u1a, u1b, _r1(c1),
                               u2, _r1(c2), d1, _r1(g1), d2, _r1(g2),
                               d3, _r1(g3), mask3)
    return pred


# speedup vs baseline: 3.1797x; 1.2911x over previous
import jax
import jax.numpy as jnp
from jax import lax
from jax.experimental import pallas as pl
from jax.experimental.pallas import tpu as pltpu
from jax.experimental.pallas import tpu_sc as plsc

_N = 10000
_E = 320000
_H = 128
_NC = 2
_NS = 16
_NW = _NC * _NS
_CH = 128
_RPT = 624
_RLAST0 = _RPT * 15
_RLAST = _N - _RLAST0
_NB = 2000
_EB = 2000

_f32 = jnp.float32


def _full(shape):
    return pl.BlockSpec(shape, lambda i: (0,) * len(shape))


def _rows(shape):
    return pl.BlockSpec(shape, lambda i: (i,) + (0,) * (len(shape) - 1))


def _relu(v):
    return jnp.maximum(v, 0.0)


def _enc_node_body(x_ref, a1, b1, a2, b2, w1a, w1b, bm1, h_out, p_out, q_out):
    t = _relu(x_ref[...] @ a1[...] + b1[...])
    h = t @ a2[...] + b2[...]
    h_out[...] = h
    p_out[...] = h @ w1a[...] + bm1[...]
    q_out[...] = h @ w1b[...]


def _enc_edge_body(ea_ref, a1, b1, a2, b2, e_out):
    t = _relu(ea_ref[...] @ a1[...] + b1[...])
    e_out[...] = t @ a2[...] + b2[...]


def _edge_body(s1_ref, s2_ref, e_ref, w1c, w2, b2, m_out):
    pre = s1_ref[...] + s2_ref[...] + e_ref[...] @ w1c[...]
    m_out[...] = _relu(pre) @ w2[...] + b2[...]


def _node_body(h_ref, a0_ref, a1_ref, u1a, u1b, c1, u2, c2, w1a, w1b, bm1,
               h_out, p_out, q_out):
    h = h_ref[...]
    agg = a0_ref[...] + a1_ref[...]
    z = _relu(h @ u1a[...] + agg @ u1b[...] + c1[...])
    hn = h + z @ u2[...] + c2[...]
    h_out[...] = hn
    p_out[...] = hn @ w1a[...] + bm1[...]
    q_out[...] = hn @ w1b[...]


def _node_final_body(h_ref, a0_ref, a1_ref, u1a, u1b, c1, u2, c2,
                     d1, g1, d2, g2, d3, g3, msk_ref, out_ref):
    h = h_ref[...]
    agg = a0_ref[...] + a1_ref[...]
    z = _relu(h @ u1a[...] + agg @ u1b[...] + c1[...])
    hn = h + z @ u2[...] + c2[...]
    z1 = _relu(hn @ d1[...] + g1[...])
    z2 = _relu(z1 @ d2[...] + g2[...])
    out_ref[...] = (z2 @ d3[...] + g3[...]) * msk_ref[...]


def _nmat(n, d):
    return jax.ShapeDtypeStruct((n, d), _f32)


def _encode_nodes(x2, a1, b1, a2, b2, w1a, w1b, bm1):
    return pl.pallas_call(
        _enc_node_body,
        grid=(_N // _NB,),
        in_specs=[_rows((_NB, x2.shape[1])), _full(a1.shape), _full(b1.shape),
                  _full(a2.shape), _full(b2.shape), _full(w1a.shape),
                  _full(w1b.shape), _full(bm1.shape)],
        out_specs=[_rows((_NB, _H))] * 3,
        out_shape=[_nmat(_N, _H)] * 3,
    )(x2, a1, b1, a2, b2, w1a, w1b, bm1)


def _encode_edges(ea, a1, b1, a2, b2):
    return pl.pallas_call(
        _enc_edge_body,
        grid=(_E // _EB,),
        in_specs=[_rows((_EB, ea.shape[1])), _full(a1.shape), _full(b1.shape),
                  _full(a2.shape), _full(b2.shape)],
        out_specs=_rows((_EB, _H)),
        out_shape=_nmat(_E, _H),
    )(ea, a1, b1, a2, b2)


def _edge_mlp(s1, s2, e, w1c, w2, b2):
    return pl.pallas_call(
        _edge_body,
        grid=(_E // _EB,),
        in_specs=[_rows((_EB, _H))] * 3 + [_full(w1c.shape), _full(w2.shape),
                                           _full(b2.shape)],
        out_specs=_rows((_EB, _H)),
        out_shape=_nmat(_E, _H),
    )(s1, s2, e, w1c, w2, b2)


def _node_update(h, a0, a1m, u1a, u1b, c1, u2, c2, w1a, w1b, bm1):
    return pl.pallas_call(
        _node_body,
        grid=(_N // _NB,),
        in_specs=[_rows((_NB, _H))] * 3 +
                 [_full(w.shape) for w in (u1a, u1b, c1, u2, c2, w1a, w1b, bm1)],
        out_specs=[_rows((_NB, _H))] * 3,
        out_shape=[_nmat(_N, _H)] * 3,
    )(h, a0, a1m, u1a, u1b, c1, u2, c2, w1a, w1b, bm1)


def _node_final(h, a0, a1m, u1a, u1b, c1, u2, c2, d1, g1, d2, g2, d3, g3, msk):
    return pl.pallas_call(
        _node_final_body,
        grid=(_N // _NB,),
        in_specs=[_rows((_NB, _H))] * 3 +
                 [_full(w.shape) for w in (u1a, u1b, c1, u2, c2,
                                           d1, g1, d2, g2, d3, g3)] +
                 [_rows((_NB, 3))],
        out_specs=_rows((_NB, 3)),
        out_shape=_nmat(_N, 3),
    )(h, a0, a1m, u1a, u1b, c1, u2, c2, d1, g1, d2, g2, d3, g3, msk)


def _sc_mesh():
    return plsc.VectorSubcoreMesh(core_axis_name="c", subcore_axis_name="s",
                                  num_cores=_NC, num_subcores=_NS)


def _worker_id():
    return lax.axis_index("s") * _NC + lax.axis_index("c")


_EPW = _E // _NW
_GF = _EPW // _CH
_GT = _EPW - _GF * _CH
_GNB = 4
_GTASKS = 2 * _GF


def _sc_gather_body(p_hbm, q_hbm, src_hbm, dst_hbm, s1_hbm, s2_hbm,
                    sidx, didx, tidx, bufs, tbuf, *sems):
    sg = sems[:_GNB]
    sw = sems[_GNB:]
    w = _worker_id()
    e0 = w * _EPW

    pltpu.sync_copy(src_hbm.at[pl.ds(e0, _GF * _CH)], sidx)
    pltpu.sync_copy(dst_hbm.at[pl.ds(e0, _GF * _CH)], didx)

    def idx_of(j, b):
        ref = sidx if b % 2 == 0 else didx
        return ref.at[pl.ds(j * _CH, _CH)]

    def tab_of(b):
        return p_hbm if b % 2 == 0 else q_hbm

    def out_of(b):
        return s1_hbm if b % 2 == 0 else s2_hbm

    def start_gather(g, b):
        j = 2 * g + b // 2
        pltpu.async_copy(tab_of(b).at[idx_of(j, b)], bufs.at[b], sg[b])

    def wait_gather(g, b):
        j = 2 * g + b // 2
        pltpu.make_async_copy(tab_of(b).at[idx_of(j, b)], bufs.at[b],
                              sg[b]).wait()

    def start_wb(g, b):
        j = 2 * g + b // 2
        pltpu.async_copy(bufs.at[b], out_of(b).at[pl.ds(e0 + j * _CH, _CH)],
                         sw[b])

    def wait_wb(g, b):
        j = 2 * g + b // 2
        pltpu.make_async_copy(bufs.at[b],
                              out_of(b).at[pl.ds(e0 + j * _CH, _CH)],
                              sw[b]).wait()

    start_gather(0, 0)
    for b in range(1, _GNB):
        wait_gather(0, b - 1)
        start_wb(0, b - 1)
        start_gather(0, b)

    def group(g, carry):
        for b in range(_GNB):
            pb = (b - 1) % _GNB
            pg = g if b > 0 else g - 1
            wait_gather(pg, pb)
            start_wb(pg, pb)
            wait_wb(g - 1, b)
            start_gather(g, b)
        return carry

    lax.fori_loop(1, _GTASKS // _GNB, group, 0)

    last = _GTASKS // _GNB - 1
    wait_gather(last, _GNB - 1)
    start_wb(last, _GNB - 1)
    for b in range(_GNB):
        wait_wb(last, b)

    pltpu.sync_copy(src_hbm.at[pl.ds(e0 + _GF * _CH, _GT)], tidx)
    pltpu.sync_copy(p_hbm.at[tidx], tbuf)
    pltpu.sync_copy(tbuf, s1_hbm.at[pl.ds(e0 + _GF * _CH, _GT)])
    pltpu.sync_copy(dst_hbm.at[pl.ds(e0 + _GF * _CH, _GT)], tidx)
    pltpu.sync_copy(q_hbm.at[tidx], tbuf)
    pltpu.sync_copy(tbuf, s2_hbm.at[pl.ds(e0 + _GF * _CH, _GT)])


def _sc_gather(p, q, src, dst):
    fn = pl.kernel(
        _sc_gather_body,
        out_type=[_nmat(_E, _H), _nmat(_E, _H)],
        mesh=_sc_mesh(),
        scratch_types=[pltpu.VMEM((_GF * _CH,), jnp.int32),
                       pltpu.VMEM((_GF * _CH,), jnp.int32),
                       pltpu.VMEM((_GT,), jnp.int32),
                       pltpu.VMEM((_GNB, _CH, _H), _f32),
                       pltpu.VMEM((_GT, _H), _f32)] +
                      [pltpu.SemaphoreType.DMA] * (2 * _GNB),
    )
    return fn(p, q, src, dst)


_SNB = 2


def _sc_scatter_body(m_hbm, dst_hbm, z_hbm, out_hbm,
                     idxs, bufs, tidx, tbuf, agg_sh, *sems):
    si = sems[:_SNB]
    sl = sems[_SNB:2 * _SNB]
    ss = sems[2 * _SNB:]
    c = lax.axis_index("c")
    s = lax.axis_index("s")
    w = s * _NC + c
    e0 = w * _EPW
    r0 = s * _RPT
    @pl.when(s < _NS - 1)
    def _():
        pltpu.sync_copy(z_hbm.at[pl.ds(r0, _RPT)], agg_sh.at[pl.ds(r0, _RPT)])

    @pl.when(s == _NS - 1)
    def _():
        pltpu.sync_copy(z_hbm.at[pl.ds(_RLAST0, _RLAST)],
                        agg_sh.at[pl.ds(_RLAST0, _RLAST)])

    plsc.subcore_barrier()

    def start_load(j, b):
        base = e0 + j * _CH
        pltpu.async_copy(dst_hbm.at[pl.ds(base, _CH)], idxs.at[b], si[b])
        pltpu.async_copy(m_hbm.at[pl.ds(base, _CH)], bufs.at[b], sl[b])

    def wait_load(j, b):
        base = e0 + j * _CH
        pltpu.make_async_copy(dst_hbm.at[pl.ds(base, _CH)], idxs.at[b],
                              si[b]).wait()
        pltpu.make_async_copy(m_hbm.at[pl.ds(base, _CH)], bufs.at[b],
                              sl[b]).wait()

    def start_sadd(b):
        pltpu.async_copy(bufs.at[b], agg_sh.at[idxs.at[b]], ss[b], add=True)

    def wait_sadd(b):
        pltpu.make_async_copy(bufs.at[b], agg_sh.at[idxs.at[b]],
                              ss[b]).wait()

    start_load(0, 0)
    for b in range(1, _SNB):
        wait_load(b - 1, b - 1)
        start_sadd(b - 1)
        start_load(b, b)

    def group(g, carry):
        for b in range(_SNB):
            t = g * _SNB + b
            pb = (b - 1) % _SNB
            wait_load(t - 1, pb)
            start_sadd(pb)
            wait_sadd(b)
            start_load(t, b)
        return carry

    lax.fori_loop(1, _GF // _SNB, group, 0)

    wait_load(_GF - 1, _SNB - 1)
    start_sadd(_SNB - 1)
    for b in range(_SNB):
        wait_sadd(b)

    pltpu.sync_copy(dst_hbm.at[pl.ds(e0 + _GF * _CH, _GT)], tidx)
    pltpu.sync_copy(m_hbm.at[pl.ds(e0 + _GF * _CH, _GT)], tbuf)
    pltpu.sync_copy(tbuf, agg_sh.at[tidx], add=True)

    plsc.subcore_barrier()

    @pl.when(s < _NS - 1)
    def _():
        pltpu.sync_copy(agg_sh.at[pl.ds(r0, _RPT)],
                        out_hbm.at[c, pl.ds(r0, _RPT)])

    @pl.when(s == _NS - 1)
    def _():
        pltpu.sync_copy(agg_sh.at[pl.ds(_RLAST0, _RLAST)],
                        out_hbm.at[c, pl.ds(_RLAST0, _RLAST)])


def _sc_scatter(m, dst, zeros_n):
    fn = pl.kernel(
        _sc_scatter_body,
        out_type=jax.ShapeDtypeStruct((_NC, _N, _H), _f32),
        mesh=_sc_mesh(),
        scratch_types=[pltpu.VMEM((_SNB, _CH), jnp.int32),
                       pltpu.VMEM((_SNB, _CH, _H), _f32),
                       pltpu.VMEM((_GT,), jnp.int32),
                       pltpu.VMEM((_GT, _H), _f32),
                       pltpu.VMEM_SHARED((_N, _H), _f32),
                       pltpu.SemaphoreType.DMA] +
                      [pltpu.SemaphoreType.DMA] * (3 * _SNB - 1),
    )
    return fn(m, dst, zeros_n)


def _r1(b):
    return b.reshape(1, -1)


def kernel(x, coords, edge_attr, bc_disp, bc_rot, edge_index,
           enc_node, enc_edge, mp_params, dec):
    x2 = jnp.concatenate([coords, x[:, 3:]], axis=1)
    src = edge_index[0]
    dst = edge_index[1]
    mask3 = jnp.concatenate([1.0 - bc_disp, 1.0 - bc_disp, 1.0 - bc_rot],
                            axis=1)
    zeros_n = jnp.zeros((_N, _H), _f32)

    (ne1, nb1), (ne2, nb2) = enc_node
    (ee1, eb1), (ee2, eb2) = enc_edge

    def _w1_split(l):
        w1, b1 = mp_params[l][0][0]
        return w1[:_H], w1[_H:2 * _H], w1[2 * _H:], b1

    e = _encode_edges(edge_attr, ee1, _r1(eb1), ee2, _r1(eb2))

    w1a, w1b, _, b1 = _w1_split(0)
    h, p, q = _encode_nodes(x2, ne1, _r1(nb1), ne2, _r1(nb2),
                            w1a, w1b, _r1(b1))

    pred = None
    for l in range(len(mp_params)):
        edge_mlp, node_mlp = mp_params[l]
        _, (w2, b2) = edge_mlp
        (u1, c1), (u2, c2) = node_mlp
        _, _, w1c, _ = _w1_split(l)

        s1, s2 = _sc_gather(p, q, src, dst)
        m = _edge_mlp(s1, s2, e, w1c, w2, _r1(b2))
        aggs = _sc_scatter(m, dst, zeros_n)

        u1a, u1b = u1[:_H], u1[_H:]
        if l + 1 < len(mp_params):
            w1a_n, w1b_n, _, b1_n = _w1_split(l + 1)
            h, p, q = _node_update(h, aggs[0], aggs[1], u1a, u1b, _r1(c1),
                                   u2, _r1(c2), w1a_n, w1b_n, _r1(b1_n))
        else:
            (d1, g1), (d2, g2), (d3, g3) = dec
            pred = _node_final(h, aggs[0], aggs[1], u1a, u1b, _r1(c1),
                               u2, _r1(c2), d1, _r1(g1), d2, _r1(g2),
                               d3, _r1(g3), mask3)
    return pred
